# speedup vs baseline: 1.1004x; 1.1004x over previous
import functools

import jax
import jax.numpy as jnp
from jax.experimental import pallas as pl
from jax.experimental.pallas import tpu as pltpu

_B, _L, _D, _O, _E, _T = 4, 2048, 1024, 1024, 8, 256
_N = _L // _T
_S = _B * _N
_SB = 8
_TO = 256
_NO = _O // _TO


def _gating_body(x_ref, wg_ref, gates_ref, loss_ref, logits_ref):
    k = pl.program_id(0)
    xm = jnp.mean(x_ref[...], axis=1)
    logits_ref[pl.ds(k * _SB, _SB), :] = jax.lax.dot_general(
        xm, wg_ref[...], (((1,), (0,)), ((), ())),
        preferred_element_type=jnp.float32)

    @pl.when(k == pl.num_programs(0) - 1)
    def _():
        logits = logits_ref[...]
        m = jnp.max(logits, axis=1, keepdims=True)
        ex = jnp.exp(logits - m)
        p = ex / jnp.sum(ex, axis=1, keepdims=True)
        imp = jnp.sum(p, axis=0, keepdims=True)
        ld = jnp.sum((p > 0).astype(jnp.float32), axis=0, keepdims=True)

        def cv2(v):
            mean = jnp.sum(v, axis=1, keepdims=True) / _E
            var = jnp.sum((v - mean) ** 2, axis=1, keepdims=True) / (_E - 1)
            return var / (mean * mean + 1e-10)

        loss_ref[...] = (cv2(imp) + cv2(ld)) * 0.001
        rolled = jnp.concatenate([p[:1], p[:-1]], axis=0)
        row = jax.lax.broadcasted_iota(jnp.int32, (_S, _E), 0)
        gates_ref[...] = jnp.where(row % _N == 0, p, rolled)


def _moe_body(g_ref, x_ref, w_ref, rw_ref, b_ref, rb_ref, out_ref,
              dwm_ref, dbm_ref):
    s = pl.program_id(1)

    @pl.when(s == 0)
    def _():
        w = w_ref[...]
        rw = rw_ref[...]
        dw = w - rw[None]
        sgn = jnp.sign(jnp.sum(dw, axis=0))
        mw = jnp.sign(dw)
        mw = mw * (mw == sgn[None]).astype(jnp.float32)
        dwm_ref[...] = dw * mw
        db = b_ref[...] - rb_ref[...]
        sgnb = jnp.sign(jnp.sum(db, axis=0, keepdims=True))
        mb = jnp.sign(db)
        mb = mb * (mb == sgnb).astype(jnp.float32)
        dbm_ref[...] = db * mb

    merged = rw_ref[...]
    mrow = rb_ref[...]
    for e in range(_E):
        ge = g_ref[s, e]
        merged = merged + ge * dwm_ref[e]
        mrow = mrow + ge * dbm_ref[pl.ds(e, 1), :]
    y = jax.lax.dot_general(
        x_ref[0], merged, (((1,), (1,)), ((), ())),
        preferred_element_type=jnp.float32)
    out_ref[0] = y + mrow


def _build_calls(interpret=False):
    gating = pl.pallas_call(
        _gating_body,
        grid=(_S // _SB,),
        in_specs=[
            pl.BlockSpec((_SB, _T, _D), lambda k: (k, 0, 0)),
            pl.BlockSpec((_D, _E), lambda k: (0, 0)),
        ],
        out_specs=[
            pl.BlockSpec((_S, _E), lambda k: (0, 0)),
            pl.BlockSpec((1, 1), lambda k: (0, 0)),
        ],
        out_shape=[
            jax.ShapeDtypeStruct((_S, _E), jnp.float32),
            jax.ShapeDtypeStruct((1, 1), jnp.float32),
        ],
        scratch_shapes=[pltpu.VMEM((_S, _E), jnp.float32)],
        interpret=interpret,
    )
    moe = pl.pallas_call(
        _moe_body,
        grid=(_NO, _S),
        in_specs=[
            pl.BlockSpec(memory_space=pltpu.SMEM),
            pl.BlockSpec((1, _T, _D), lambda no, s: (s, 0, 0)),
            pl.BlockSpec((_E, _TO, _D), lambda no, s: (0, no, 0)),
            pl.BlockSpec((_TO, _D), lambda no, s: (no, 0)),
            pl.BlockSpec((_E, _TO), lambda no, s: (0, no)),
            pl.BlockSpec((1, _TO), lambda no, s: (0, no)),
        ],
        out_specs=pl.BlockSpec((1, _T, _TO), lambda no, s: (s, 0, no)),
        out_shape=jax.ShapeDtypeStruct((_S, _T, _TO * _NO), jnp.float32),
        scratch_shapes=[
            pltpu.VMEM((_E, _TO, _D), jnp.float32),
            pltpu.VMEM((_E, _TO), jnp.float32),
        ],
        interpret=interpret,
    )
    return gating, moe


_GATING, _MOE = _build_calls()


def kernel(x, w_gate, weight, bias, res_weight, res_bias):
    xc = x.reshape(_S, _T, _D)
    gates, loss = _GATING(xc, w_gate)
    out = _MOE(gates, xc, weight, res_weight, bias, res_bias)
    return out.reshape(_B, _L, _O), loss[0, 0]


# speedup vs baseline: 1.4610x; 1.3278x over previous
import functools

import jax
import jax.numpy as jnp
from jax.experimental import pallas as pl
from jax.experimental.pallas import tpu as pltpu

_B, _L, _D, _O, _E, _T = 4, 2048, 1024, 1024, 8, 256
_N = _L // _T
_S = _B * _N
_SB = 8
_TP = 256
_OO = 256


def _gating_body(x_ref, wg_ref, gates_ref, loss_ref, logits_ref):
    k = pl.program_id(0)
    xm = jnp.mean(x_ref[...], axis=1)
    logits_ref[pl.ds(k * _SB, _SB), :] = jax.lax.dot_general(
        xm, wg_ref[...], (((1,), (0,)), ((), ())),
        preferred_element_type=jnp.float32)

    @pl.when(k == pl.num_programs(0) - 1)
    def _():
        logits = logits_ref[...]
        m = jnp.max(logits, axis=1, keepdims=True)
        ex = jnp.exp(logits - m)
        p = ex / jnp.sum(ex, axis=1, keepdims=True)
        imp = jnp.sum(p, axis=0, keepdims=True)
        ld = jnp.sum((p > 0).astype(jnp.float32), axis=0, keepdims=True)

        def cv2(v):
            mean = jnp.sum(v, axis=1, keepdims=True) / _E
            var = jnp.sum((v - mean) ** 2, axis=1, keepdims=True) / (_E - 1)
            return var / (mean * mean + 1e-10)

        loss_ref[...] = (cv2(imp) + cv2(ld)) * 0.001
        rolled = jnp.concatenate([p[:1], p[:-1]], axis=0)
        row = jax.lax.broadcasted_iota(jnp.int32, (_S, _E), 0)
        gates_ref[...] = jnp.where(row % _N == 0, p, rolled)


def _ties_body(w_ref, rw_ref, b_ref, rb_ref, dwm_ref, dbm_ref):
    w = w_ref[...]
    rw = rw_ref[...]
    dw = w - rw[None]
    sgn = jnp.sign(jnp.sum(dw, axis=0))
    mw = jnp.sign(dw)
    mw = mw * (mw == sgn[None]).astype(jnp.float32)
    dwm_ref[...] = (dw * mw).astype(jnp.bfloat16)
    db = b_ref[...] - rb_ref[...]
    sgnb = jnp.sign(jnp.sum(db, axis=0, keepdims=True))
    mb = jnp.sign(db)
    mb = mb * (mb == sgnb).astype(jnp.float32)
    dbm_ref[...] = db * mb


def _moe_body(g_ref, x_ref, dwm_ref, rw_ref, dbm_ref, rb_ref, out_ref):
    s = pl.program_id(0)
    xb = x_ref[0].astype(jnp.bfloat16)
    mrow = rb_ref[...]
    for e in range(_E):
        mrow = mrow + g_ref[s, e] * dbm_ref[pl.ds(e, 1), :]
    ys = []
    for oo in range(_O // _OO):
        merged = rw_ref[pl.ds(oo * _OO, _OO), :]
        for e in range(_E):
            ge = g_ref[s, e].astype(jnp.bfloat16)
            merged = merged + ge * dwm_ref[e, pl.ds(oo * _OO, _OO), :]
        ys.append(jax.lax.dot_general(
            xb, merged, (((1,), (1,)), ((), ())),
            preferred_element_type=jnp.float32))
    out_ref[0] = jnp.concatenate(ys, axis=1) + mrow


def _build_calls(interpret=False):
    gating = pl.pallas_call(
        _gating_body,
        grid=(_S // _SB,),
        in_specs=[
            pl.BlockSpec((_SB, _T, _D), lambda k: (k, 0, 0)),
            pl.BlockSpec((_D, _E), lambda k: (0, 0)),
        ],
        out_specs=[
            pl.BlockSpec((_S, _E), lambda k: (0, 0)),
            pl.BlockSpec((1, 1), lambda k: (0, 0)),
        ],
        out_shape=[
            jax.ShapeDtypeStruct((_S, _E), jnp.float32),
            jax.ShapeDtypeStruct((1, 1), jnp.float32),
        ],
        scratch_shapes=[pltpu.VMEM((_S, _E), jnp.float32)],
        interpret=interpret,
    )
    ties = pl.pallas_call(
        _ties_body,
        grid=(_O // _TP,),
        in_specs=[
            pl.BlockSpec((_E, _TP, _D), lambda j: (0, j, 0)),
            pl.BlockSpec((_TP, _D), lambda j: (j, 0)),
            pl.BlockSpec((_E, _TP), lambda j: (0, j)),
            pl.BlockSpec((1, _TP), lambda j: (0, j)),
        ],
        out_specs=[
            pl.BlockSpec((_E, _TP, _D), lambda j: (0, j, 0)),
            pl.BlockSpec((_E, _TP), lambda j: (0, j)),
        ],
        out_shape=[
            jax.ShapeDtypeStruct((_E, _O, _D), jnp.bfloat16),
            jax.ShapeDtypeStruct((_E, _O), jnp.float32),
        ],
        interpret=interpret,
    )
    moe = pl.pallas_call(
        _moe_body,
        grid=(_S,),
        in_specs=[
            pl.BlockSpec(memory_space=pltpu.SMEM),
            pl.BlockSpec((1, _T, _D), lambda s: (s, 0, 0)),
            pl.BlockSpec((_E, _O, _D), lambda s: (0, 0, 0)),
            pl.BlockSpec((_O, _D), lambda s: (0, 0)),
            pl.BlockSpec((_E, _O), lambda s: (0, 0)),
            pl.BlockSpec((1, _O), lambda s: (0, 0)),
        ],
        out_specs=pl.BlockSpec((1, _T, _O), lambda s: (s, 0, 0)),
        out_shape=jax.ShapeDtypeStruct((_S, _T, _O), jnp.float32),
        interpret=interpret,
    )
    return gating, ties, moe


_GATING, _TIES, _MOE = _build_calls()


def kernel(x, w_gate, weight, bias, res_weight, res_bias):
    xc = x.reshape(_S, _T, _D)
    gates, loss = _GATING(xc, w_gate)
    dwm, dbm = _TIES(weight, res_weight, bias, res_bias)
    rw_bf = res_weight.astype(jnp.bfloat16)
    out = _MOE(gates, xc, dwm, rw_bf, dbm, res_bias)
    return out.reshape(_B, _L, _O), loss[0, 0]


# speedup vs baseline: 2.0955x; 1.4343x over previous
import functools

import jax
import jax.numpy as jnp
from jax.experimental import pallas as pl
from jax.experimental.pallas import tpu as pltpu

_B, _L, _D, _O, _E, _T = 4, 2048, 1024, 1024, 8, 256
_N = _L // _T
_S = _B * _N
_SB = 8
_TP = 256
_SG = 4
_OO = 256


def _gating_body(x_ref, wg_ref, gates_ref, loss_ref, xbf_ref, logits_ref):
    k = pl.program_id(0)
    xbf_ref[...] = x_ref[...].astype(jnp.bfloat16)
    xm = jnp.mean(x_ref[...], axis=1)
    logits_ref[pl.ds(k * _SB, _SB), :] = jax.lax.dot_general(
        xm, wg_ref[...], (((1,), (0,)), ((), ())),
        preferred_element_type=jnp.float32)

    @pl.when(k == pl.num_programs(0) - 1)
    def _():
        logits = logits_ref[...]
        m = jnp.max(logits, axis=1, keepdims=True)
        ex = jnp.exp(logits - m)
        p = ex / jnp.sum(ex, axis=1, keepdims=True)
        imp = jnp.sum(p, axis=0, keepdims=True)
        ld = jnp.sum((p > 0).astype(jnp.float32), axis=0, keepdims=True)

        def cv2(v):
            mean = jnp.sum(v, axis=1, keepdims=True) / _E
            var = jnp.sum((v - mean) ** 2, axis=1, keepdims=True) / (_E - 1)
            return var / (mean * mean + 1e-10)

        loss_ref[...] = (cv2(imp) + cv2(ld)) * 0.001
        rolled = jnp.concatenate([p[:1], p[:-1]], axis=0)
        row = jax.lax.broadcasted_iota(jnp.int32, (_S, _E), 0)
        gates_ref[...] = jnp.where(row % _N == 0, p, rolled)


def _ties_body(w_ref, rw_ref, b_ref, rb_ref, wt_ref, dbm_ref):
    w = w_ref[...]
    rw = rw_ref[...]
    dw = w - rw[None]
    sdw = jnp.sum(dw, axis=0)
    dwm = jnp.where(dw * sdw[None] > 0, jnp.abs(dw), 0.0)
    wt_ref[...] = (rw[None] + dwm).astype(jnp.bfloat16)
    db = b_ref[...] - rb_ref[...]
    sdb = jnp.sum(db, axis=0, keepdims=True)
    dbm_ref[...] = jnp.where(db * sdb > 0, jnp.abs(db), 0.0)


def _moe_body(g_ref, x_ref, wt_ref, dbm_ref, rb_ref, out_ref):
    k = pl.program_id(0)
    gbf = [[g_ref[k * _SG + j, e].astype(jnp.bfloat16) for e in range(_E)]
           for j in range(_SG)]
    mrows = []
    for j in range(_SG):
        mrow = rb_ref[...]
        for e in range(_E):
            mrow = mrow + g_ref[k * _SG + j, e] * dbm_ref[pl.ds(e, 1), :]
        mrows.append(mrow)
    ys = [[] for _ in range(_SG)]
    for oo in range(_O // _OO):
        sl = pl.ds(oo * _OO, _OO)
        we = [wt_ref[e, sl, :] for e in range(_E)]
        for j in range(_SG):
            merged = gbf[j][0] * we[0]
            for e in range(1, _E):
                merged = merged + gbf[j][e] * we[e]
            ys[j].append(jax.lax.dot_general(
                x_ref[j], merged, (((1,), (1,)), ((), ())),
                preferred_element_type=jnp.float32))
    for j in range(_SG):
        out_ref[j] = jnp.concatenate(ys[j], axis=1) + mrows[j]


def _build_calls(interpret=False):
    gating = pl.pallas_call(
        _gating_body,
        grid=(_S // _SB,),
        in_specs=[
            pl.BlockSpec((_SB, _T, _D), lambda k: (k, 0, 0)),
            pl.BlockSpec((_D, _E), lambda k: (0, 0)),
        ],
        out_specs=[
            pl.BlockSpec((_S, _E), lambda k: (0, 0)),
            pl.BlockSpec((1, 1), lambda k: (0, 0)),
            pl.BlockSpec((_SB, _T, _D), lambda k: (k, 0, 0)),
        ],
        out_shape=[
            jax.ShapeDtypeStruct((_S, _E), jnp.float32),
            jax.ShapeDtypeStruct((1, 1), jnp.float32),
            jax.ShapeDtypeStruct((_S, _T, _D), jnp.bfloat16),
        ],
        scratch_shapes=[pltpu.VMEM((_S, _E), jnp.float32)],
        interpret=interpret,
    )
    ties = pl.pallas_call(
        _ties_body,
        grid=(_O // _TP,),
        in_specs=[
            pl.BlockSpec((_E, _TP, _D), lambda j: (0, j, 0)),
            pl.BlockSpec((_TP, _D), lambda j: (j, 0)),
            pl.BlockSpec((_E, _TP), lambda j: (0, j)),
            pl.BlockSpec((1, _TP), lambda j: (0, j)),
        ],
        out_specs=[
            pl.BlockSpec((_E, _TP, _D), lambda j: (0, j, 0)),
            pl.BlockSpec((_E, _TP), lambda j: (0, j)),
        ],
        out_shape=[
            jax.ShapeDtypeStruct((_E, _O, _D), jnp.bfloat16),
            jax.ShapeDtypeStruct((_E, _O), jnp.float32),
        ],
        interpret=interpret,
    )
    moe = pl.pallas_call(
        _moe_body,
        grid=(_S // _SG,),
        in_specs=[
            pl.BlockSpec(memory_space=pltpu.SMEM),
            pl.BlockSpec((_SG, _T, _D), lambda k: (k, 0, 0)),
            pl.BlockSpec((_E, _O, _D), lambda k: (0, 0, 0)),
            pl.BlockSpec((_E, _O), lambda k: (0, 0)),
            pl.BlockSpec((1, _O), lambda k: (0, 0)),
        ],
        out_specs=pl.BlockSpec((_SG, _T, _O), lambda k: (k, 0, 0)),
        out_shape=jax.ShapeDtypeStruct((_S, _T, _O), jnp.float32),
        interpret=interpret,
    )
    return gating, ties, moe


_GATING, _TIES, _MOE = _build_calls()


def kernel(x, w_gate, weight, bias, res_weight, res_bias):
    xc = x.reshape(_S, _T, _D)
    gates, loss, xbf = _GATING(xc, w_gate)
    wt, dbm = _TIES(weight, res_weight, bias, res_bias)
    out = _MOE(gates, xbf, wt, dbm, res_bias)
    return out.reshape(_B, _L, _O), loss[0, 0]


# speedup vs baseline: 2.2236x; 1.0611x over previous
import functools

import jax
import jax.numpy as jnp
from jax.experimental import pallas as pl
from jax.experimental.pallas import tpu as pltpu

_B, _L, _D, _O, _E, _T = 4, 2048, 1024, 1024, 8, 256
_N = _L // _T
_S = _B * _N
_SB = 8
_TP = 256
_SG = 4
_OO = 256


def _gating_body(x_ref, wg_ref, gates_ref, loss_ref, xbf_ref, logits_ref):
    k = pl.program_id(0)
    xbf_ref[...] = x_ref[...].astype(jnp.bfloat16)
    xm = jnp.mean(x_ref[...], axis=1)
    logits_ref[pl.ds(k * _SB, _SB), :] = jax.lax.dot_general(
        xm, wg_ref[...], (((1,), (0,)), ((), ())),
        preferred_element_type=jnp.float32)

    @pl.when(k == pl.num_programs(0) - 1)
    def _():
        logits = logits_ref[...]
        m = jnp.max(logits, axis=1, keepdims=True)
        ex = jnp.exp(logits - m)
        p = ex / jnp.sum(ex, axis=1, keepdims=True)
        imp = jnp.sum(p, axis=0, keepdims=True)
        ld = jnp.sum((p > 0).astype(jnp.float32), axis=0, keepdims=True)

        def cv2(v):
            mean = jnp.sum(v, axis=1, keepdims=True) / _E
            var = jnp.sum((v - mean) ** 2, axis=1, keepdims=True) / (_E - 1)
            return var / (mean * mean + 1e-10)

        loss_ref[...] = (cv2(imp) + cv2(ld)) * 0.001
        rolled = jnp.concatenate([p[:1], p[:-1]], axis=0)
        row = jax.lax.broadcasted_iota(jnp.int32, (_S, _E), 0)
        gates_ref[...] = jnp.where(row % _N == 0, p, rolled)


def _ties_body(w_ref, rw_ref, b_ref, rb_ref, wt_ref, dbm_ref):
    w = w_ref[...]
    rw = rw_ref[...]
    dw = w - rw[None]
    sdw = jnp.sum(dw, axis=0)
    dwm = jnp.where(dw * sdw[None] > 0, jnp.abs(dw), 0.0)
    wt_ref[...] = (rw[None] + dwm).astype(jnp.bfloat16)
    db = b_ref[...] - rb_ref[...]
    sdb = jnp.sum(db, axis=0, keepdims=True)
    dbm_ref[...] = jnp.where(db * sdb > 0, jnp.abs(db), 0.0)


_OH = _O // 2
_NM = _N - 1


def _moe_body(g_ref, x_ref, wt_ref, dbm_ref, rb_ref, out_ref):
    b = pl.program_id(1)

    def xs(i):
        if i == 0:
            return x_ref[pl.ds(0, 2)].reshape(2 * _T, _D)
        return x_ref[i + 1]
    g = [[g_ref[_N * b + (0 if i == 0 else i + 1), e] for e in range(_E)]
         for i in range(_NM)]
    gbf = [[v.astype(jnp.bfloat16) for v in row] for row in g]
    mrows = []
    for i in range(_NM):
        mr = rb_ref[...]
        for e in range(_E):
            mr = mr + g[i][e] * dbm_ref[pl.ds(e, 1), :]
        mrows.append(mr)
    pieces = []
    for oo in range(_OH // _OO):
        sl = pl.ds(oo * _OO, _OO)
        we = [wt_ref[e, sl, :] for e in range(_E)]
        ys = []
        for i in range(_NM):
            merged = gbf[i][0] * we[0]
            for e in range(1, _E):
                merged = merged + gbf[i][e] * we[e]
            ys.append(jax.lax.dot_general(
                xs(i), merged, (((1,), (1,)), ((), ())),
                preferred_element_type=jnp.float32))
        pieces.append(jnp.concatenate(ys, axis=0))
    y = jnp.concatenate(pieces, axis=1)
    bias_full = jnp.concatenate([mrows[0]] * 2 + mrows[1:], axis=0)
    out_ref[...] = y.reshape(_N, _T, _OH) + bias_full[:, None, :]


def _build_calls(interpret=False):
    gating = pl.pallas_call(
        _gating_body,
        grid=(_S // _SB,),
        in_specs=[
            pl.BlockSpec((_SB, _T, _D), lambda k: (k, 0, 0)),
            pl.BlockSpec((_D, _E), lambda k: (0, 0)),
        ],
        out_specs=[
            pl.BlockSpec((_S, _E), lambda k: (0, 0)),
            pl.BlockSpec((1, 1), lambda k: (0, 0)),
            pl.BlockSpec((_SB, _T, _D), lambda k: (k, 0, 0)),
        ],
        out_shape=[
            jax.ShapeDtypeStruct((_S, _E), jnp.float32),
            jax.ShapeDtypeStruct((1, 1), jnp.float32),
            jax.ShapeDtypeStruct((_S, _T, _D), jnp.bfloat16),
        ],
        scratch_shapes=[pltpu.VMEM((_S, _E), jnp.float32)],
        interpret=interpret,
    )
    ties = pl.pallas_call(
        _ties_body,
        grid=(_O // _TP,),
        in_specs=[
            pl.BlockSpec((_E, _TP, _D), lambda j: (0, j, 0)),
            pl.BlockSpec((_TP, _D), lambda j: (j, 0)),
            pl.BlockSpec((_E, _TP), lambda j: (0, j)),
            pl.BlockSpec((1, _TP), lambda j: (0, j)),
        ],
        out_specs=[
            pl.BlockSpec((_E, _TP, _D), lambda j: (0, j, 0)),
            pl.BlockSpec((_E, _TP), lambda j: (0, j)),
        ],
        out_shape=[
            jax.ShapeDtypeStruct((_E, _O, _D), jnp.bfloat16),
            jax.ShapeDtypeStruct((_E, _O), jnp.float32),
        ],
        interpret=interpret,
    )
    moe = pl.pallas_call(
        _moe_body,
        grid=(_O // _OH, _B),
        in_specs=[
            pl.BlockSpec(memory_space=pltpu.SMEM),
            pl.BlockSpec((_N, _T, _D), lambda oh, b: (b, 0, 0)),
            pl.BlockSpec((_E, _OH, _D), lambda oh, b: (0, oh, 0)),
            pl.BlockSpec((_E, _OH), lambda oh, b: (0, oh)),
            pl.BlockSpec((1, _OH), lambda oh, b: (0, oh)),
        ],
        out_specs=pl.BlockSpec((_N, _T, _OH), lambda oh, b: (b, 0, oh)),
        out_shape=jax.ShapeDtypeStruct((_S, _T, _O), jnp.float32),
        interpret=interpret,
    )
    return gating, ties, moe


_GATING, _TIES, _MOE = _build_calls()


def kernel(x, w_gate, weight, bias, res_weight, res_bias):
    xc = x.reshape(_S, _T, _D)
    gates, loss, xbf = _GATING(xc, w_gate)
    wt, dbm = _TIES(weight, res_weight, bias, res_bias)
    out = _MOE(gates, xbf, wt, dbm, res_bias)
    return out.reshape(_B, _L, _O), loss[0, 0]
